# baseline (speedup 1.0000x reference)
import functools

import jax
import jax.numpy as jnp
import numpy as np
from jax import lax
from jax.experimental import pallas as pl
from jax.experimental.pallas import tpu as pltpu
from jax.experimental.pallas import tpu_sc as plsc

RADII = (0.1, 0.3, 0.5)
KNN = 32
_B, _N = 8, 1024
_INV_SQRT_BN = 1.0 / (1.0 + 1e-3) ** 0.5

_NC, _NS = 2, 16
_NW = _NC * _NS
_RPW = (_B * _N) // _NW
_GRP = 16
_NGRP = _RPW // _GRP


def _sqrt_le_threshold(r: float) -> float:
    r32 = np.float32(r)
    z = np.float32(r32 * r32)
    while np.sqrt(np.float32(np.nextafter(z, np.float32(np.inf)))) <= r32:
        z = np.float32(np.nextafter(z, np.float32(np.inf)))
    while np.sqrt(z) > r32:
        z = np.float32(np.nextafter(z, np.float32(-np.inf)))
    return float(z)


def _threefry2x32(k0, k1, x0, x1):
    rot = ((13, 15, 26, 6), (17, 29, 16, 24))
    ks = (np.uint32(k0), np.uint32(k1),
          np.uint32(k0) ^ np.uint32(k1) ^ np.uint32(0x1BD11BDA))
    x0 = (x0 + ks[0]).astype(np.uint32)
    x1 = (x1 + ks[1]).astype(np.uint32)

    def rotl(v, d):
        return ((v << np.uint32(d)) | (v >> np.uint32(32 - d))).astype(np.uint32)

    for i in range(5):
        for r in rot[i % 2]:
            x0 = (x0 + x1).astype(np.uint32)
            x1 = rotl(x1, r)
            x1 = x1 ^ x0
        x0 = (x0 + ks[(i + 1) % 3]).astype(np.uint32)
        x1 = (x1 + ks[(i + 2) % 3] + np.uint32(i + 1)).astype(np.uint32)
    return x0, x1


def _uniform_key42(shape):
    size = int(np.prod(shape))
    counts = np.arange(size, dtype=np.uint64)
    h0 = (counts >> np.uint64(32)).astype(np.uint32)
    h1 = (counts & np.uint64(0xFFFFFFFF)).astype(np.uint32)
    o0, o1 = _threefry2x32(0, 42, h0, h1)
    bits = o0 ^ o1
    floats = ((bits >> np.uint32(9)) | np.uint32(0x3F800000)).view(np.float32)
    return (floats - np.float32(1.0)).reshape(shape)


def _build_noise_tables():
    n = _uniform_key42((_B, _N, _N))
    perm = np.argsort(-n, axis=-1, kind="stable").astype(np.int32)
    nz = (_N - (n == 0.0).sum(axis=-1)).astype(np.int32)
    zx = np.full((_B, _N), -1, dtype=np.int32)
    zb, zi, zj = np.nonzero(n == 0.0)
    zx[zb, zi] = zj
    perm = perm.reshape(_B * _N // _GRP, _GRP * _N)
    return perm, nz.reshape(-1), zx.reshape(-1)


_PERM, _NZ, _ZX = _build_noise_tables()
_T2 = tuple(_sqrt_le_threshold(r) for r in RADII)


def _tnet_body(x_ref, tW1_ref, tb1_ref, tg1_ref, tB1_ref, tW2_ref, tb2_ref,
               tg2_ref, tB2_ref, tW3_ref, tb3_ref, pct_ref):
    scale = jnp.float32(_INV_SQRT_BN)
    x = x_ref[0]
    h = jnp.dot(x, tW1_ref[...], preferred_element_type=jnp.float32) + tb1_ref[...]
    h = jax.nn.relu(tg1_ref[...] * h * scale + tB1_ref[...])
    m = jnp.max(h, axis=0, keepdims=True)
    h2 = jnp.dot(m, tW2_ref[...], preferred_element_type=jnp.float32) + tb2_ref[...]
    h2 = jax.nn.relu(tg2_ref[...] * h2 * scale + tB2_ref[...])
    t = jnp.dot(h2, tW3_ref[...], preferred_element_type=jnp.float32) + tb3_ref[...]
    T = jnp.concatenate([t[:, 0:3], t[:, 3:6], t[:, 6:9]], axis=0)
    pct = jnp.dot(x, T, preferred_element_type=jnp.float32)
    pct_ref[0] = pct


def _tnet(inputs, tW1, tb1, tg1, tB1, tW2, tb2, tg2, tB2, tW3, tb3):
    row = lambda v: v.reshape(1, -1)
    args = (row(tb1), row(tg1), row(tB1), tW2, row(tb2), row(tg2), row(tB2),
            tW3, row(tb3))
    full = lambda a: pl.BlockSpec(a.shape, lambda b: (0,) * a.ndim)
    return pl.pallas_call(
        _tnet_body,
        grid=(_B,),
        in_specs=[pl.BlockSpec((1, _N, 3), lambda b: (b, 0, 0)), full(tW1)]
                 + [full(a) for a in args],
        out_specs=pl.BlockSpec((1, _N, 3), lambda b: (b, 0, 0)),
        out_shape=jax.ShapeDtypeStruct((_B, _N, 3), jnp.float32),
    )(inputs, tW1, *args)


def _sc_body(pct_hbm, perm_hbm, nz_hbm, zx_hbm, out_hbm,
             px_v, py_v, pz_v, d2_v, perm_v, frow_v, nz_v, zx_v,
             buf0, buf1, cidx, cd2):
    wid = lax.axis_index("s") * _NC + lax.axis_index("c")
    base = wid * _RPW
    batch = base // _N
    pltpu.sync_copy(pct_hbm.at[3 * batch], px_v.at[pl.ds(0, _N)])
    pltpu.sync_copy(pct_hbm.at[3 * batch + 1], py_v.at[pl.ds(0, _N)])
    pltpu.sync_copy(pct_hbm.at[3 * batch + 2], pz_v.at[pl.ds(0, _N)])
    pltpu.sync_copy(nz_hbm.at[pl.ds(base, _RPW)], nz_v.at[pl.ds(0, _RPW)])
    pltpu.sync_copy(zx_hbm.at[pl.ds(base, _RPW)], zx_v.at[pl.ds(0, _RPW)])
    iota = lax.iota(jnp.int32, 16)
    t0, t1, t2 = (jnp.float32(t) for t in _T2)

    def count(m):
        return plsc.all_reduce_population_count(m)

    def append(ref, p, x, m, limit):
        pos = jnp.minimum(p + plsc.cumsum(m.astype(jnp.int32)) - 1, limit)
        plsc.store_scatter(ref, [pos], x, mask=m)

    def group(g, _):
        grp = wid * _NGRP + g
        pltpu.sync_copy(perm_hbm.at[grp], perm_v)

        def rowfn(j, _):
            local = lax.rem(base, _N) + g * _GRP + j
            qx = px_v[pl.ds(local, 16)][0]
            qy = py_v[pl.ds(local, 16)][0]
            qz = pz_v[pl.ds(local, 16)][0]
            nzc = nz_v[pl.ds(g * _GRP + j, 16)][0]
            zid = zx_v[pl.ds(g * _GRP + j, 16)][0]

            def sweep(k, _):
                sl = pl.ds(k * 16, 16)
                dx = px_v[sl] - qx
                dy = py_v[sl] - qy
                dz = pz_v[sl] - qz
                d2_v[sl] = dx * dx + dy * dy + dz * dz
                return ()

            lax.fori_loop(0, _N // 16, sweep, (), unroll=4)

            zero_v = iota - iota

            def tier1(k, p):
                idxv = perm_v[pl.ds(j * _N + k * 16, 16)]
                d2 = plsc.load_gather(d2_v, [idxv])
                m2 = d2 <= t2
                pos = p + plsc.cumsum(m2.astype(jnp.int32)) - 1
                plsc.store_scatter(cidx, [pos], idxv, mask=m2)
                plsc.store_scatter(cd2, [pos], d2, mask=m2)
                return p + count(m2)

            hv = lax.fori_loop(0, _N // 16 - 1, tier1, zero_v, unroll=2)
            lastv = perm_v[pl.ds(j * _N + _N - 16, 16)]
            lastd = plsc.load_gather(d2_v, [lastv])
            m2 = (lastd <= t2) & ((_N - 16 + iota) < nzc)
            pos = hv + plsc.cumsum(m2.astype(jnp.int32)) - 1
            plsc.store_scatter(cidx, [pos], lastv, mask=m2)
            plsc.store_scatter(cd2, [pos], lastd, mask=m2)
            hv = hv + count(m2)
            hits = hv[0]

            def tier2(k, ptrs):
                p0, p1 = ptrs
                idxv = cidx[pl.ds(k * 16, 16)]
                d2 = cd2[pl.ds(k * 16, 16)]
                valid = (k * 16 + iota) < hv
                m0 = (d2 <= t0) & valid
                m1 = (d2 <= t1) & valid
                append(buf0, p0, idxv, m0, 63)
                append(buf1, p1, idxv, m1, 63)
                return (p0 + count(m0), p1 + count(m1))

            nchunks = (hits + 15) // 16
            ptrs01 = lax.fori_loop(0, nchunks, tier2, (zero_v, zero_v))

            def fill(k, ptrs):
                p0, p1, p2 = ptrs
                col = k * 16 + iota
                d2 = d2_v[pl.ds(k * 16, 16)]
                zm = col == zid
                m0 = (d2 > t0) | zm
                m1 = (d2 > t1) | zm
                m2 = (d2 > t2) | zm
                append(buf0, p0, col, m0, 63)
                append(buf1, p1, col, m1, 63)
                append(cidx, p2, col, m2, _N + 15)
                return (p0 + count(m0), p1 + count(m1), p2 + count(m2))

            lax.fori_loop(0, 4, fill, (ptrs01[0], ptrs01[1], hv), unroll=2)

            fbase = j * (3 * KNN * 3)
            for r, buf in enumerate((buf0, buf1, cidx)):
                for hh in range(KNN // 16):
                    idxv = buf[pl.ds(16 * hh, 16)]
                    posv = fbase + 9 * (iota + 16 * hh) + 3 * r
                    for cc, pv in enumerate((px_v, py_v, pz_v)):
                        val = plsc.load_gather(pv, [idxv])
                        plsc.store_scatter(frow_v, [posv + cc], val)
            return ()

        lax.fori_loop(0, _GRP, rowfn, ())
        pltpu.sync_copy(frow_v, out_hbm.at[grp])
        return ()

    lax.fori_loop(0, _NGRP, group, ())


def _sc_select_gather(pct_flat):
    mesh = plsc.VectorSubcoreMesh(core_axis_name="c", subcore_axis_name="s",
                                  num_cores=_NC, num_subcores=_NS)
    fn = pl.kernel(
        _sc_body,
        out_type=jax.ShapeDtypeStruct((_B * _N // _GRP, _GRP * 3 * KNN * 3),
                                      jnp.float32),
        mesh=mesh,
        compiler_params=pltpu.CompilerParams(needs_layout_passes=False),
        scratch_types=[
            pltpu.VMEM((_N + 16,), jnp.float32),
            pltpu.VMEM((_N + 16,), jnp.float32),
            pltpu.VMEM((_N + 16,), jnp.float32),
            pltpu.VMEM((_N + 16,), jnp.float32),
            pltpu.VMEM((_GRP * _N,), jnp.int32),
            pltpu.VMEM((_GRP * 3 * KNN * 3,), jnp.float32),
            pltpu.VMEM((_RPW + 16,), jnp.int32),
            pltpu.VMEM((_RPW + 16,), jnp.int32),
            pltpu.VMEM((64,), jnp.int32),
            pltpu.VMEM((64,), jnp.int32),
            pltpu.VMEM((_N + 16,), jnp.int32),
            pltpu.VMEM((_N + 16,), jnp.float32),
        ],
    )
    return fn(pct_flat, jnp.asarray(_PERM), jnp.asarray(_NZ), jnp.asarray(_ZX))


def _stack_body(feats_ref, Wc1_ref, bc1_ref, g1_ref, B1_ref, Wres_ref, bres_ref,
                Wk1_ref, bk1_ref, gk1_ref, Bk1_ref, Wk2_ref, bk2_ref, gk2_ref, Bk2_ref,
                Wk3_ref, bk3_ref, gk3_ref, Bk3_ref, out_ref):
    b = pl.program_id(0)
    nb = pl.program_id(1)
    f = feats_ref[0]
    scale = jnp.float32(_INV_SQRT_BN)

    def dense_bn_relu(x, W, bias, g, B):
        h = jnp.dot(x, W[...], preferred_element_type=jnp.float32) + bias[...]
        return jax.nn.relu(g[...] * h * scale + B[...])

    f = dense_bn_relu(f, Wc1_ref, bc1_ref, g1_ref, B1_ref)
    f = f + jnp.dot(f, Wres_ref[...], preferred_element_type=jnp.float32) + bres_ref[...]
    f = dense_bn_relu(f, Wk1_ref, bk1_ref, gk1_ref, Bk1_ref)
    f = dense_bn_relu(f, Wk2_ref, bk2_ref, gk2_ref, Bk2_ref)
    f = dense_bn_relu(f, Wk3_ref, bk3_ref, gk3_ref, Bk3_ref)
    bmax = jnp.max(f, axis=0, keepdims=True)

    @pl.when(nb == 0)
    def _():
        out_ref[pl.ds(b, 1), :] = bmax

    @pl.when(nb != 0)
    def _():
        out_ref[pl.ds(b, 1), :] = jnp.maximum(out_ref[pl.ds(b, 1), :], bmax)


def _dense_stack(feats, Wc1, bc1, g1, B1, Wres, bres,
                 Wk1, bk1, gk1, Bk1, Wk2, bk2, gk2, Bk2, Wk3, bk3, gk3, Bk3):
    B, N, F = feats.shape
    RB = 256
    grid = (B, N // RB)
    row = lambda v: v.reshape(1, -1)
    full = lambda a: pl.BlockSpec(a.shape, lambda b, n: (0,) * a.ndim)
    args = (Wc1, row(bc1), row(g1), row(B1), Wres, row(bres),
            Wk1, row(bk1), row(gk1), row(Bk1), Wk2, row(bk2), row(gk2), row(Bk2),
            Wk3, row(bk3), row(gk3), row(Bk3))
    return pl.pallas_call(
        _stack_body,
        grid=grid,
        in_specs=[pl.BlockSpec((1, RB, F), lambda b, n: (b, n, 0))] + [full(a) for a in args],
        out_specs=pl.BlockSpec((B, 170), lambda b, n: (0, 0)),
        out_shape=jax.ShapeDtypeStruct((B, 170), jnp.float32),
    )(feats, *args)


def kernel(inputs, tW1, tb1, tg1, tB1, tW2, tb2, tg2, tB2, tW3, tb3,
           Wc1, bc1, g1, B1, Wres, bres,
           Wk1, bk1, gk1, Bk1, Wk2, bk2, gk2, Bk2, Wk3, bk3, gk3, Bk3):
    pct = _tnet(inputs, tW1, tb1, tg1, tB1, tW2, tb2, tg2, tB2, tW3, tb3)
    pct_soa = jnp.transpose(pct, (0, 2, 1)).reshape(3 * _B, _N)
    feats = _sc_select_gather(pct_soa)
    feats = feats.reshape(_B, _N, 3 * KNN * 3)
    return _dense_stack(feats, Wc1, bc1, g1, B1, Wres, bres,
                        Wk1, bk1, gk1, Bk1, Wk2, bk2, gk2, Bk2, Wk3, bk3, gk3, Bk3)


# speedup vs baseline: 1.1490x; 1.1490x over previous
import functools

import jax
import jax.numpy as jnp
import numpy as np
from jax import lax
from jax.experimental import pallas as pl
from jax.experimental.pallas import tpu as pltpu
from jax.experimental.pallas import tpu_sc as plsc

RADII = (0.1, 0.3, 0.5)
KNN = 32
_B, _N = 8, 1024
_INV_SQRT_BN = 1.0 / (1.0 + 1e-3) ** 0.5

_NC, _NS = 2, 16
_NW = _NC * _NS
_RPW = (_B * _N) // _NW
_GRP = 16
_NGRP = _RPW // _GRP


def _sqrt_le_threshold(r: float) -> float:
    r32 = np.float32(r)
    z = np.float32(r32 * r32)
    while np.sqrt(np.float32(np.nextafter(z, np.float32(np.inf)))) <= r32:
        z = np.float32(np.nextafter(z, np.float32(np.inf)))
    while np.sqrt(z) > r32:
        z = np.float32(np.nextafter(z, np.float32(-np.inf)))
    return float(z)


def _threefry2x32(k0, k1, x0, x1):
    rot = ((13, 15, 26, 6), (17, 29, 16, 24))
    ks = (np.uint32(k0), np.uint32(k1),
          np.uint32(k0) ^ np.uint32(k1) ^ np.uint32(0x1BD11BDA))
    x0 = (x0 + ks[0]).astype(np.uint32)
    x1 = (x1 + ks[1]).astype(np.uint32)

    def rotl(v, d):
        return ((v << np.uint32(d)) | (v >> np.uint32(32 - d))).astype(np.uint32)

    for i in range(5):
        for r in rot[i % 2]:
            x0 = (x0 + x1).astype(np.uint32)
            x1 = rotl(x1, r)
            x1 = x1 ^ x0
        x0 = (x0 + ks[(i + 1) % 3]).astype(np.uint32)
        x1 = (x1 + ks[(i + 2) % 3] + np.uint32(i + 1)).astype(np.uint32)
    return x0, x1


def _uniform_key42(shape):
    size = int(np.prod(shape))
    counts = np.arange(size, dtype=np.uint64)
    h0 = (counts >> np.uint64(32)).astype(np.uint32)
    h1 = (counts & np.uint64(0xFFFFFFFF)).astype(np.uint32)
    o0, o1 = _threefry2x32(0, 42, h0, h1)
    bits = o0 ^ o1
    floats = ((bits >> np.uint32(9)) | np.uint32(0x3F800000)).view(np.float32)
    return (floats - np.float32(1.0)).reshape(shape)


def _build_noise_tables():
    n = _uniform_key42((_B, _N, _N))
    perm = np.argsort(-n, axis=-1, kind="stable").astype(np.int32)
    nz = (_N - (n == 0.0).sum(axis=-1)).astype(np.int32)
    zx = np.full((_B, _N), -1, dtype=np.int32)
    zb, zi, zj = np.nonzero(n == 0.0)
    zx[zb, zi] = zj
    perm = perm.reshape(_B * _N // _GRP, _GRP * _N)
    return perm, nz.reshape(-1), zx.reshape(-1)


_PERM, _NZ, _ZX = _build_noise_tables()
_T2 = tuple(_sqrt_le_threshold(r) for r in RADII)


def _tnet_body(x_ref, tW1_ref, tb1_ref, tg1_ref, tB1_ref, tW2_ref, tb2_ref,
               tg2_ref, tB2_ref, tW3_ref, tb3_ref, pct_ref):
    scale = jnp.float32(_INV_SQRT_BN)
    x = x_ref[0]
    h = jnp.dot(x, tW1_ref[...], preferred_element_type=jnp.float32) + tb1_ref[...]
    h = jax.nn.relu(tg1_ref[...] * h * scale + tB1_ref[...])
    m = jnp.max(h, axis=0, keepdims=True)
    h2 = jnp.dot(m, tW2_ref[...], preferred_element_type=jnp.float32) + tb2_ref[...]
    h2 = jax.nn.relu(tg2_ref[...] * h2 * scale + tB2_ref[...])
    t = jnp.dot(h2, tW3_ref[...], preferred_element_type=jnp.float32) + tb3_ref[...]
    T = jnp.concatenate([t[:, 0:3], t[:, 3:6], t[:, 6:9]], axis=0)
    pct = jnp.dot(x, T, preferred_element_type=jnp.float32)
    pct_ref[0] = pct


def _tnet(inputs, tW1, tb1, tg1, tB1, tW2, tb2, tg2, tB2, tW3, tb3):
    row = lambda v: v.reshape(1, -1)
    args = (row(tb1), row(tg1), row(tB1), tW2, row(tb2), row(tg2), row(tB2),
            tW3, row(tb3))
    full = lambda a: pl.BlockSpec(a.shape, lambda b: (0,) * a.ndim)
    return pl.pallas_call(
        _tnet_body,
        grid=(_B,),
        in_specs=[pl.BlockSpec((1, _N, 3), lambda b: (b, 0, 0)), full(tW1)]
                 + [full(a) for a in args],
        out_specs=pl.BlockSpec((1, _N, 3), lambda b: (b, 0, 0)),
        out_shape=jax.ShapeDtypeStruct((_B, _N, 3), jnp.float32),
    )(inputs, tW1, *args)


def _sc_body(pct_hbm, perm_hbm, nz_hbm, zx_hbm, out_hbm,
             px_v, py_v, pz_v, d2_v, perm_v, frow_v, nz_v, zx_v,
             buf0, buf1, buf2, ci0, ci1, ci2, ci3, cd0, cd1, cd2_, cd3):
    cidx = (ci0, ci1, ci2, ci3)
    cd2 = (cd0, cd1, cd2_, cd3)
    wid = lax.axis_index("s") * _NC + lax.axis_index("c")
    base = wid * _RPW
    batch = base // _N
    pltpu.sync_copy(pct_hbm.at[3 * batch], px_v.at[pl.ds(0, _N)])
    pltpu.sync_copy(pct_hbm.at[3 * batch + 1], py_v.at[pl.ds(0, _N)])
    pltpu.sync_copy(pct_hbm.at[3 * batch + 2], pz_v.at[pl.ds(0, _N)])
    pltpu.sync_copy(nz_hbm.at[pl.ds(base, _RPW)], nz_v.at[pl.ds(0, _RPW)])
    pltpu.sync_copy(zx_hbm.at[pl.ds(base, _RPW)], zx_v.at[pl.ds(0, _RPW)])
    iota = lax.iota(jnp.int32, 16)
    t0, t1, t2 = (jnp.float32(t) for t in _T2)

    def count(m):
        return plsc.all_reduce_population_count(m)[0]

    def group(g, _):
        grp = wid * _NGRP + g
        pltpu.sync_copy(perm_hbm.at[grp], perm_v)

        def rowfn(j, _):
            local = lax.rem(base, _N) + g * _GRP + j
            qx = px_v[pl.ds(local, 16)][0]
            qy = py_v[pl.ds(local, 16)][0]
            qz = pz_v[pl.ds(local, 16)][0]
            nzc = nz_v[pl.ds(g * _GRP + j, 16)][0]
            zid = zx_v[pl.ds(g * _GRP + j, 16)][0]

            def sweep(k, _):
                sl = pl.ds(k * 16, 16)
                dx = px_v[sl] - qx
                dy = py_v[sl] - qy
                dz = pz_v[sl] - qz
                d2_v[sl] = dx * dx + dy * dy + dz * dz
                return ()

            lax.fori_loop(0, _N // 16, sweep, (), unroll=4)

            def t1chunk(cbase, p, ci, cd):
                idxv = perm_v[pl.ds(j * _N + cbase, 16)]
                d2 = plsc.load_gather(d2_v, [idxv])
                m2 = d2 <= t2
                plsc.store_compressed(ci.at[pl.ds(p, 16)], idxv, mask=m2)
                plsc.store_compressed(cd.at[pl.ds(p, 16)], d2, mask=m2)
                return p + count(m2)

            qn = _N // 4

            def tier1(k, ps):
                return tuple(
                    t1chunk(q * qn + k * 16, ps[q], cidx[q], cd2[q])
                    for q in range(4))

            hq = list(lax.fori_loop(0, qn // 16 - 1, tier1, (0, 0, 0, 0),
                                    unroll=2))
            for q in range(3):
                hq[q] = t1chunk(q * qn + qn - 16, hq[q], cidx[q], cd2[q])
            lastv = perm_v[pl.ds(j * _N + _N - 16, 16)]
            lastd = plsc.load_gather(d2_v, [lastv])
            m2 = (lastd <= t2) & ((_N - 16 + iota) < nzc)
            plsc.store_compressed(cidx[3].at[pl.ds(hq[3], 16)], lastv, mask=m2)
            plsc.store_compressed(cd2[3].at[pl.ds(hq[3], 16)], lastd, mask=m2)
            hq[3] = hq[3] + count(m2)

            def t2body(ci, cd, nq):
                def go(k, ptrs):
                    p0, p1, p2 = ptrs
                    idxv = ci[pl.ds(k * 16, 16)]
                    d2 = cd[pl.ds(k * 16, 16)]
                    valid = (k * 16 + iota) < nq
                    m0 = (d2 <= t0) & valid
                    m1 = (d2 <= t1) & valid
                    m2 = valid
                    plsc.store_compressed(
                        buf0.at[pl.ds(jnp.minimum(p0, KNN), 16)], idxv, mask=m0)
                    plsc.store_compressed(
                        buf1.at[pl.ds(jnp.minimum(p1, KNN), 16)], idxv, mask=m1)
                    plsc.store_compressed(
                        buf2.at[pl.ds(jnp.minimum(p2, KNN), 16)], idxv, mask=m2)
                    return (p0 + count(m0), p1 + count(m1), p2 + count(m2))
                return go

            ptrs = (0, 0, 0)
            for q in range(4):
                ptrs = lax.fori_loop(0, (hq[q] + 15) // 16,
                                     t2body(cidx[q], cd2[q], hq[q]), ptrs)

            def fill(k, ptrs):
                p0, p1, p2 = ptrs
                col = k * 16 + iota
                d2 = d2_v[pl.ds(k * 16, 16)]
                zm = col == zid
                m0 = (d2 > t0) | zm
                m1 = (d2 > t1) | zm
                m2 = (d2 > t2) | zm
                plsc.store_compressed(
                    buf0.at[pl.ds(jnp.minimum(p0, KNN), 16)], col, mask=m0)
                plsc.store_compressed(
                    buf1.at[pl.ds(jnp.minimum(p1, KNN), 16)], col, mask=m1)
                plsc.store_compressed(
                    buf2.at[pl.ds(jnp.minimum(p2, KNN), 16)], col, mask=m2)
                return (p0 + count(m0), p1 + count(m1), p2 + count(m2))

            lax.fori_loop(0, 4, fill, ptrs, unroll=2)

            fbase = j * (3 * KNN * 3)
            for r, buf in enumerate((buf0, buf1, buf2)):
                for hh in range(KNN // 16):
                    idxv = buf[pl.ds(16 * hh, 16)]
                    posv = fbase + 9 * (iota + 16 * hh) + 3 * r
                    for cc, pv in enumerate((px_v, py_v, pz_v)):
                        val = plsc.load_gather(pv, [idxv])
                        plsc.store_scatter(frow_v, [posv + cc], val)
            return ()

        lax.fori_loop(0, _GRP, rowfn, ())
        pltpu.sync_copy(frow_v, out_hbm.at[grp])
        return ()

    lax.fori_loop(0, _NGRP, group, ())


def _sc_select_gather(pct_flat):
    mesh = plsc.VectorSubcoreMesh(core_axis_name="c", subcore_axis_name="s",
                                  num_cores=_NC, num_subcores=_NS)
    fn = pl.kernel(
        _sc_body,
        out_type=jax.ShapeDtypeStruct((_B * _N // _GRP, _GRP * 3 * KNN * 3),
                                      jnp.float32),
        mesh=mesh,
        compiler_params=pltpu.CompilerParams(needs_layout_passes=False),
        scratch_types=[
            pltpu.VMEM((_N + 16,), jnp.float32),
            pltpu.VMEM((_N + 16,), jnp.float32),
            pltpu.VMEM((_N + 16,), jnp.float32),
            pltpu.VMEM((_N + 16,), jnp.float32),
            pltpu.VMEM((_GRP * _N,), jnp.int32),
            pltpu.VMEM((_GRP * 3 * KNN * 3,), jnp.float32),
            pltpu.VMEM((_RPW + 16,), jnp.int32),
            pltpu.VMEM((_RPW + 16,), jnp.int32),
            pltpu.VMEM((64,), jnp.int32),
            pltpu.VMEM((64,), jnp.int32),
            pltpu.VMEM((64,), jnp.int32),
        ] + [pltpu.VMEM((_N // 4 + 16,), jnp.int32) for _ in range(4)]
          + [pltpu.VMEM((_N // 4 + 16,), jnp.float32) for _ in range(4)],
    )
    return fn(pct_flat, jnp.asarray(_PERM), jnp.asarray(_NZ), jnp.asarray(_ZX))


def _stack_body(feats_ref, Wc1_ref, bc1_ref, g1_ref, B1_ref, Wres_ref, bres_ref,
                Wk1_ref, bk1_ref, gk1_ref, Bk1_ref, Wk2_ref, bk2_ref, gk2_ref, Bk2_ref,
                Wk3_ref, bk3_ref, gk3_ref, Bk3_ref, out_ref):
    b = pl.program_id(0)
    nb = pl.program_id(1)
    f = feats_ref[0]
    scale = jnp.float32(_INV_SQRT_BN)

    def dense_bn_relu(x, W, bias, g, B):
        h = jnp.dot(x, W[...], preferred_element_type=jnp.float32) + bias[...]
        return jax.nn.relu(g[...] * h * scale + B[...])

    f = dense_bn_relu(f, Wc1_ref, bc1_ref, g1_ref, B1_ref)
    f = f + jnp.dot(f, Wres_ref[...], preferred_element_type=jnp.float32) + bres_ref[...]
    f = dense_bn_relu(f, Wk1_ref, bk1_ref, gk1_ref, Bk1_ref)
    f = dense_bn_relu(f, Wk2_ref, bk2_ref, gk2_ref, Bk2_ref)
    f = dense_bn_relu(f, Wk3_ref, bk3_ref, gk3_ref, Bk3_ref)
    bmax = jnp.max(f, axis=0, keepdims=True)

    @pl.when(nb == 0)
    def _():
        out_ref[pl.ds(b, 1), :] = bmax

    @pl.when(nb != 0)
    def _():
        out_ref[pl.ds(b, 1), :] = jnp.maximum(out_ref[pl.ds(b, 1), :], bmax)


def _dense_stack(feats, Wc1, bc1, g1, B1, Wres, bres,
                 Wk1, bk1, gk1, Bk1, Wk2, bk2, gk2, Bk2, Wk3, bk3, gk3, Bk3):
    B, N, F = feats.shape
    RB = 256
    grid = (B, N // RB)
    row = lambda v: v.reshape(1, -1)
    full = lambda a: pl.BlockSpec(a.shape, lambda b, n: (0,) * a.ndim)
    args = (Wc1, row(bc1), row(g1), row(B1), Wres, row(bres),
            Wk1, row(bk1), row(gk1), row(Bk1), Wk2, row(bk2), row(gk2), row(Bk2),
            Wk3, row(bk3), row(gk3), row(Bk3))
    return pl.pallas_call(
        _stack_body,
        grid=grid,
        in_specs=[pl.BlockSpec((1, RB, F), lambda b, n: (b, n, 0))] + [full(a) for a in args],
        out_specs=pl.BlockSpec((B, 170), lambda b, n: (0, 0)),
        out_shape=jax.ShapeDtypeStruct((B, 170), jnp.float32),
    )(feats, *args)


def kernel(inputs, tW1, tb1, tg1, tB1, tW2, tb2, tg2, tB2, tW3, tb3,
           Wc1, bc1, g1, B1, Wres, bres,
           Wk1, bk1, gk1, Bk1, Wk2, bk2, gk2, Bk2, Wk3, bk3, gk3, Bk3):
    pct = _tnet(inputs, tW1, tb1, tg1, tB1, tW2, tb2, tg2, tB2, tW3, tb3)
    pct_soa = jnp.transpose(pct, (0, 2, 1)).reshape(3 * _B, _N)
    feats = _sc_select_gather(pct_soa)
    feats = feats.reshape(_B, _N, 3 * KNN * 3)
    return _dense_stack(feats, Wc1, bc1, g1, B1, Wres, bres,
                        Wk1, bk1, gk1, Bk1, Wk2, bk2, gk2, Bk2, Wk3, bk3, gk3, Bk3)


# speedup vs baseline: 1.1949x; 1.0399x over previous
import functools

import jax
import jax.numpy as jnp
import numpy as np
from jax import lax
from jax.experimental import pallas as pl
from jax.experimental.pallas import tpu as pltpu
from jax.experimental.pallas import tpu_sc as plsc

RADII = (0.1, 0.3, 0.5)
KNN = 32
_B, _N = 8, 1024
_INV_SQRT_BN = 1.0 / (1.0 + 1e-3) ** 0.5

_NC, _NS = 2, 16
_NW = _NC * _NS
_RPW = (_B * _N) // _NW
_GRP = 16
_NGRP = _RPW // _GRP


def _sqrt_le_threshold(r: float) -> float:
    r32 = np.float32(r)
    z = np.float32(r32 * r32)
    while np.sqrt(np.float32(np.nextafter(z, np.float32(np.inf)))) <= r32:
        z = np.float32(np.nextafter(z, np.float32(np.inf)))
    while np.sqrt(z) > r32:
        z = np.float32(np.nextafter(z, np.float32(-np.inf)))
    return float(z)


def _threefry2x32(k0, k1, x0, x1):
    rot = ((13, 15, 26, 6), (17, 29, 16, 24))
    ks = (np.uint32(k0), np.uint32(k1),
          np.uint32(k0) ^ np.uint32(k1) ^ np.uint32(0x1BD11BDA))
    x0 = (x0 + ks[0]).astype(np.uint32)
    x1 = (x1 + ks[1]).astype(np.uint32)

    def rotl(v, d):
        return ((v << np.uint32(d)) | (v >> np.uint32(32 - d))).astype(np.uint32)

    for i in range(5):
        for r in rot[i % 2]:
            x0 = (x0 + x1).astype(np.uint32)
            x1 = rotl(x1, r)
            x1 = x1 ^ x0
        x0 = (x0 + ks[(i + 1) % 3]).astype(np.uint32)
        x1 = (x1 + ks[(i + 2) % 3] + np.uint32(i + 1)).astype(np.uint32)
    return x0, x1


def _uniform_key42(shape):
    size = int(np.prod(shape))
    counts = np.arange(size, dtype=np.uint64)
    h0 = (counts >> np.uint64(32)).astype(np.uint32)
    h1 = (counts & np.uint64(0xFFFFFFFF)).astype(np.uint32)
    o0, o1 = _threefry2x32(0, 42, h0, h1)
    bits = o0 ^ o1
    floats = ((bits >> np.uint32(9)) | np.uint32(0x3F800000)).view(np.float32)
    return (floats - np.float32(1.0)).reshape(shape)


def _build_noise_tables():
    n = _uniform_key42((_B, _N, _N))
    perm = np.argsort(-n, axis=-1, kind="stable").astype(np.int32)
    prank = np.empty_like(perm)
    np.put_along_axis(prank, perm,
                      np.broadcast_to(np.arange(_N, dtype=np.int32), perm.shape),
                      axis=-1)
    nz = (_N - (n == 0.0).sum(axis=-1)).astype(np.int32)
    zx = np.full((_B, _N), -1, dtype=np.int32)
    zb, zi, zj = np.nonzero(n == 0.0)
    zx[zb, zi] = zj
    perm = perm.reshape(_B * _N // _GRP, _GRP * _N)
    prank = prank.reshape(_B * _N // _GRP, _GRP * _N)
    return perm, prank, nz.reshape(-1), zx.reshape(-1)


_PERM, _PRANK, _NZ, _ZX = _build_noise_tables()
_T2 = tuple(_sqrt_le_threshold(r) for r in RADII)


def _tnet_body(x_ref, tW1_ref, tb1_ref, tg1_ref, tB1_ref, tW2_ref, tb2_ref,
               tg2_ref, tB2_ref, tW3_ref, tb3_ref, pct_ref):
    scale = jnp.float32(_INV_SQRT_BN)
    x = x_ref[0]
    h = jnp.dot(x, tW1_ref[...], preferred_element_type=jnp.float32) + tb1_ref[...]
    h = jax.nn.relu(tg1_ref[...] * h * scale + tB1_ref[...])
    m = jnp.max(h, axis=0, keepdims=True)
    h2 = jnp.dot(m, tW2_ref[...], preferred_element_type=jnp.float32) + tb2_ref[...]
    h2 = jax.nn.relu(tg2_ref[...] * h2 * scale + tB2_ref[...])
    t = jnp.dot(h2, tW3_ref[...], preferred_element_type=jnp.float32) + tb3_ref[...]
    T = jnp.concatenate([t[:, 0:3], t[:, 3:6], t[:, 6:9]], axis=0)
    pct = jnp.dot(x, T, preferred_element_type=jnp.float32)
    pct_ref[0] = pct


def _tnet(inputs, tW1, tb1, tg1, tB1, tW2, tb2, tg2, tB2, tW3, tb3):
    row = lambda v: v.reshape(1, -1)
    args = (row(tb1), row(tg1), row(tB1), tW2, row(tb2), row(tg2), row(tB2),
            tW3, row(tb3))
    full = lambda a: pl.BlockSpec(a.shape, lambda b: (0,) * a.ndim)
    return pl.pallas_call(
        _tnet_body,
        grid=(_B,),
        in_specs=[pl.BlockSpec((1, _N, 3), lambda b: (b, 0, 0)), full(tW1)]
                 + [full(a) for a in args],
        out_specs=pl.BlockSpec((1, _N, 3), lambda b: (b, 0, 0)),
        out_shape=jax.ShapeDtypeStruct((_B, _N, 3), jnp.float32),
    )(inputs, tW1, *args)


def _sc_body(pct_hbm, perm_hbm, prank_hbm, nz_hbm, zx_hbm, out_hbm,
             px_v, py_v, pz_v, d2p_v, perm_v, prank_v, frow_v, nz_v, zx_v,
             buf0, buf1, cidx, cd2):
    wid = lax.axis_index("s") * _NC + lax.axis_index("c")
    base = wid * _RPW
    batch = base // _N
    pltpu.sync_copy(pct_hbm.at[3 * batch], px_v.at[pl.ds(0, _N)])
    pltpu.sync_copy(pct_hbm.at[3 * batch + 1], py_v.at[pl.ds(0, _N)])
    pltpu.sync_copy(pct_hbm.at[3 * batch + 2], pz_v.at[pl.ds(0, _N)])
    pltpu.sync_copy(nz_hbm.at[pl.ds(base, _RPW)], nz_v.at[pl.ds(0, _RPW)])
    pltpu.sync_copy(zx_hbm.at[pl.ds(base, _RPW)], zx_v.at[pl.ds(0, _RPW)])
    iota = lax.iota(jnp.int32, 16)
    t0, t1, t2 = (jnp.float32(t) for t in _T2)

    def count(m):
        return plsc.all_reduce_population_count(m)[0]

    def group(g, _):
        grp = wid * _NGRP + g
        pltpu.sync_copy(perm_hbm.at[grp], perm_v)
        pltpu.sync_copy(prank_hbm.at[grp], prank_v)

        def rowfn(j, _):
            local = lax.rem(base, _N) + g * _GRP + j
            qx = px_v[pl.ds(local, 16)][0]
            qy = py_v[pl.ds(local, 16)][0]
            qz = pz_v[pl.ds(local, 16)][0]
            nzc = nz_v[pl.ds(g * _GRP + j, 16)][0]
            zid = zx_v[pl.ds(g * _GRP + j, 16)][0]

            def d2chunk(k):
                sl = pl.ds(k * 16, 16)
                dx = px_v[sl] - qx
                dy = py_v[sl] - qy
                dz = pz_v[sl] - qz
                return dx * dx + dy * dy + dz * dz

            def sweep(k, _):
                pr = prank_v[pl.ds(j * _N + k * 16, 16)]
                plsc.store_scatter(d2p_v, [pr], d2chunk(k))
                return ()

            lax.fori_loop(0, _N // 16, sweep, (), unroll=4)

            def tier1(k, p):
                idxv = perm_v[pl.ds(j * _N + k * 16, 16)]
                d2 = d2p_v[pl.ds(k * 16, 16)]
                m2 = d2 <= t2
                plsc.store_compressed(cidx.at[pl.ds(p, 16)], idxv, mask=m2)
                plsc.store_compressed(cd2.at[pl.ds(p, 16)], d2, mask=m2)
                return p + count(m2)

            hits = lax.fori_loop(0, _N // 16 - 1, tier1, 0, unroll=2)
            lastv = perm_v[pl.ds(j * _N + _N - 16, 16)]
            lastd = d2p_v[pl.ds(_N - 16, 16)]
            m2 = (lastd <= t2) & ((_N - 16 + iota) < nzc)
            plsc.store_compressed(cidx.at[pl.ds(hits, 16)], lastv, mask=m2)
            plsc.store_compressed(cd2.at[pl.ds(hits, 16)], lastd, mask=m2)
            hits = hits + count(m2)

            def tier2(k, ptrs):
                p0, p1 = ptrs
                idxv = cidx[pl.ds(k * 16, 16)]
                d2 = cd2[pl.ds(k * 16, 16)]
                valid = (k * 16 + iota) < hits
                m0 = (d2 <= t0) & valid
                m1 = (d2 <= t1) & valid
                plsc.store_compressed(buf0.at[pl.ds(jnp.minimum(p0, KNN), 16)],
                                      idxv, mask=m0)
                plsc.store_compressed(buf1.at[pl.ds(jnp.minimum(p1, KNN), 16)],
                                      idxv, mask=m1)
                return (p0 + count(m0), p1 + count(m1))

            ptrs01 = lax.fori_loop(0, (hits + 15) // 16, tier2, (0, 0))
            ptrs = (ptrs01[0], ptrs01[1], hits)

            def fill(k, ptrs):
                p0, p1, p2 = ptrs
                col = k * 16 + iota
                d2 = d2chunk(k)
                zm = col == zid
                m0 = (d2 > t0) | zm
                m1 = (d2 > t1) | zm
                m2 = (d2 > t2) | zm
                plsc.store_compressed(buf0.at[pl.ds(jnp.minimum(p0, KNN), 16)],
                                      col, mask=m0)
                plsc.store_compressed(buf1.at[pl.ds(jnp.minimum(p1, KNN), 16)],
                                      col, mask=m1)
                plsc.store_compressed(cidx.at[pl.ds(jnp.minimum(p2, KNN), 16)],
                                      col, mask=m2)
                return (p0 + count(m0), p1 + count(m1), p2 + count(m2))

            lax.fori_loop(0, 4, fill, ptrs, unroll=2)

            fbase = j * (3 * KNN * 3)
            for r, buf in enumerate((buf0, buf1, cidx)):
                for hh in range(KNN // 16):
                    idxv = buf[pl.ds(16 * hh, 16)]
                    posv = fbase + 9 * (iota + 16 * hh) + 3 * r
                    for cc, pv in enumerate((px_v, py_v, pz_v)):
                        val = plsc.load_gather(pv, [idxv])
                        plsc.store_scatter(frow_v, [posv + cc], val)
            return ()

        lax.fori_loop(0, _GRP, rowfn, ())
        pltpu.sync_copy(frow_v, out_hbm.at[grp])
        return ()

    lax.fori_loop(0, _NGRP, group, ())


def _sc_select_gather(pct_flat):
    mesh = plsc.VectorSubcoreMesh(core_axis_name="c", subcore_axis_name="s",
                                  num_cores=_NC, num_subcores=_NS)
    fn = pl.kernel(
        _sc_body,
        out_type=jax.ShapeDtypeStruct((_B * _N // _GRP, _GRP * 3 * KNN * 3),
                                      jnp.float32),
        mesh=mesh,
        compiler_params=pltpu.CompilerParams(needs_layout_passes=False),
        scratch_types=[
            pltpu.VMEM((_N + 16,), jnp.float32),
            pltpu.VMEM((_N + 16,), jnp.float32),
            pltpu.VMEM((_N + 16,), jnp.float32),
            pltpu.VMEM((_N + 16,), jnp.float32),
            pltpu.VMEM((_GRP * _N,), jnp.int32),
            pltpu.VMEM((_GRP * _N,), jnp.int32),
            pltpu.VMEM((_GRP * 3 * KNN * 3,), jnp.float32),
            pltpu.VMEM((_RPW + 16,), jnp.int32),
            pltpu.VMEM((_RPW + 16,), jnp.int32),
            pltpu.VMEM((64,), jnp.int32),
            pltpu.VMEM((64,), jnp.int32),
            pltpu.VMEM((_N + 16,), jnp.int32),
            pltpu.VMEM((_N + 16,), jnp.float32),
        ],
    )
    return fn(pct_flat, jnp.asarray(_PERM), jnp.asarray(_PRANK),
              jnp.asarray(_NZ), jnp.asarray(_ZX))


def _stack_body(feats_ref, Wc1_ref, bc1_ref, g1_ref, B1_ref, Wres_ref, bres_ref,
                Wk1_ref, bk1_ref, gk1_ref, Bk1_ref, Wk2_ref, bk2_ref, gk2_ref, Bk2_ref,
                Wk3_ref, bk3_ref, gk3_ref, Bk3_ref, out_ref):
    b = pl.program_id(0)
    nb = pl.program_id(1)
    f = feats_ref[0]
    scale = jnp.float32(_INV_SQRT_BN)

    def dense_bn_relu(x, W, bias, g, B):
        h = jnp.dot(x, W[...], preferred_element_type=jnp.float32) + bias[...]
        return jax.nn.relu(g[...] * h * scale + B[...])

    f = dense_bn_relu(f, Wc1_ref, bc1_ref, g1_ref, B1_ref)
    f = f + jnp.dot(f, Wres_ref[...], preferred_element_type=jnp.float32) + bres_ref[...]
    f = dense_bn_relu(f, Wk1_ref, bk1_ref, gk1_ref, Bk1_ref)
    f = dense_bn_relu(f, Wk2_ref, bk2_ref, gk2_ref, Bk2_ref)
    f = dense_bn_relu(f, Wk3_ref, bk3_ref, gk3_ref, Bk3_ref)
    bmax = jnp.max(f, axis=0, keepdims=True)

    @pl.when(nb == 0)
    def _():
        out_ref[pl.ds(b, 1), :] = bmax

    @pl.when(nb != 0)
    def _():
        out_ref[pl.ds(b, 1), :] = jnp.maximum(out_ref[pl.ds(b, 1), :], bmax)


def _dense_stack(feats, Wc1, bc1, g1, B1, Wres, bres,
                 Wk1, bk1, gk1, Bk1, Wk2, bk2, gk2, Bk2, Wk3, bk3, gk3, Bk3):
    B, N, F = feats.shape
    RB = 256
    grid = (B, N // RB)
    row = lambda v: v.reshape(1, -1)
    full = lambda a: pl.BlockSpec(a.shape, lambda b, n: (0,) * a.ndim)
    args = (Wc1, row(bc1), row(g1), row(B1), Wres, row(bres),
            Wk1, row(bk1), row(gk1), row(Bk1), Wk2, row(bk2), row(gk2), row(Bk2),
            Wk3, row(bk3), row(gk3), row(Bk3))
    return pl.pallas_call(
        _stack_body,
        grid=grid,
        in_specs=[pl.BlockSpec((1, RB, F), lambda b, n: (b, n, 0))] + [full(a) for a in args],
        out_specs=pl.BlockSpec((B, 170), lambda b, n: (0, 0)),
        out_shape=jax.ShapeDtypeStruct((B, 170), jnp.float32),
    )(feats, *args)


def kernel(inputs, tW1, tb1, tg1, tB1, tW2, tb2, tg2, tB2, tW3, tb3,
           Wc1, bc1, g1, B1, Wres, bres,
           Wk1, bk1, gk1, Bk1, Wk2, bk2, gk2, Bk2, Wk3, bk3, gk3, Bk3):
    pct = _tnet(inputs, tW1, tb1, tg1, tB1, tW2, tb2, tg2, tB2, tW3, tb3)
    pct_soa = jnp.transpose(pct, (0, 2, 1)).reshape(3 * _B, _N)
    feats = _sc_select_gather(pct_soa)
    feats = feats.reshape(_B, _N, 3 * KNN * 3)
    return _dense_stack(feats, Wc1, bc1, g1, B1, Wres, bres,
                        Wk1, bk1, gk1, Bk1, Wk2, bk2, gk2, Bk2, Wk3, bk3, gk3, Bk3)
